# baseline (speedup 1.0000x reference)
import functools

import jax
import jax.numpy as jnp
import numpy as np
from jax import lax
from jax.experimental import pallas as pl
from jax.experimental.pallas import tpu as pltpu
from jax.experimental.pallas import tpu_sc as plsc

_LEFT_HAND = np.arange(468, 489)
_LEFT_POSE = np.array([502, 504, 506, 508, 510])
_LIPS = np.array([
    61, 185, 40, 39, 37, 0, 267, 269, 270, 409, 291, 146, 91, 181, 84, 17,
    314, 405, 321, 375, 78, 191, 80, 81, 82, 13, 312, 311, 310, 415, 95, 88,
    178, 87, 14, 317, 402, 318, 324, 308,
])
_LM_LEFT = np.concatenate((_LIPS, _LEFT_HAND, _LEFT_POSE))
_RIGHT_HAND = np.arange(522, 543)
_RIGHT_POSE = np.array([503, 505, 507, 509, 511])
_LM_RIGHT = np.concatenate((_LIPS, _RIGHT_HAND, _RIGHT_POSE))

N_FRAMES = 4096
N_LM = 543
N_OUT_LM = 66
OUT_F = 64
NS = 16
L = 16
FPT = N_FRAMES // NS
CF = 64
NGATHER = 9
OUT_W = NGATHER * L
FPT3 = OUT_F // NS


def _flat_idx(lm):
    fi = np.stack([2 * lm, 2 * lm + 1], axis=1).reshape(-1)
    return np.pad(fi, (0, OUT_W - fi.size)).astype(np.int32)


_LIDX_L = _flat_idx(_LM_LEFT)
_LIDX_R = _flat_idx(_LM_RIGHT)

_MUL_R = np.ones(OUT_W, np.float32)
_ADD_R = np.zeros(OUT_W, np.float32)
for _l in range(40, N_OUT_LM):
    _MUL_R[2 * _l] = -1.0
    _ADD_R[2 * _l] = 1.0


@functools.cache
def _build_sc_kernel():
    mesh = plsc.VectorSubcoreMesh(
        core_axis_name="c", subcore_axis_name="s", num_cores=2,
        num_subcores=NS,
    )

    @functools.partial(
        pl.kernel,
        out_type=[
            jax.ShapeDtypeStruct((OUT_F * OUT_W,), jnp.float32),
            jax.ShapeDtypeStruct((N_FRAMES,), jnp.int32),
        ],
        mesh=mesh,
        compiler_params=pltpu.CompilerParams(
            needs_layout_passes=False, use_tc_tiling_on_sc=False),
        scratch_types=[
            pltpu.VMEM((CF, 75, 2), jnp.float32),
            pltpu.VMEM((FPT,), jnp.int32),
            pltpu.VMEM((FPT,), jnp.int32),
            pltpu.VMEM((L,), jnp.int32),
            pltpu.VMEM((L,), jnp.int32),
            pltpu.VMEM_SHARED((N_FRAMES,), jnp.int32),
            pltpu.VMEM_SHARED((N_FRAMES,), jnp.int32),
            pltpu.VMEM_SHARED((NS, L), jnp.int32),
            pltpu.VMEM_SHARED((NS, L), jnp.int32),
            pltpu.VMEM_SHARED((OUT_F,), jnp.int32),
            pltpu.VMEM_SHARED((L,), jnp.int32),
            pltpu.VMEM((N_FRAMES,), jnp.int32),
            pltpu.VMEM((N_FRAMES,), jnp.int32),
            pltpu.VMEM((NS, L), jnp.int32),
            pltpu.VMEM((NS, L), jnp.int32),
            pltpu.VMEM((N_FRAMES,), jnp.int32),
            pltpu.VMEM((OUT_F,), jnp.int32),
            pltpu.VMEM((1, N_LM, 2), jnp.float32),
            pltpu.VMEM((FPT3 * OUT_W,), jnp.float32),
            pltpu.VMEM((OUT_W,), jnp.int32),
            pltpu.VMEM((OUT_W,), jnp.int32),
            pltpu.VMEM((OUT_W,), jnp.float32),
            pltpu.VMEM((OUT_W,), jnp.float32),
            pltpu.VMEM((OUT_W,), jnp.int32),
            pltpu.VMEM((OUT_W,), jnp.float32),
            pltpu.VMEM((OUT_W,), jnp.float32),
        ],
    )
    def _sc_kernel(
        x_hbm, lidxL_hbm, lidxR_hbm, mulR_hbm, addR_hbm,
        out1_hbm, oidx_hbm,
        hbuf, cntL_l, cntR_l, stageL, stageR,
        sh_cntL, sh_cntR, sh_totL, sh_totR, sh_idx, sh_ld,
        cntL_all, cntR_all, totL_all, totR_all,
        idx_buf, idx64, fbuf, obuf,
        lidxL_v, lidxR_v, mulR_v, addR_v, lidx_sel, mul_sel, add_sel,
    ):
        c = lax.axis_index("c")
        s = lax.axis_index("s")

        @pl.when(c == 0)
        def _core0():
            lane = lax.iota(jnp.int32, L)

            def p1chunk(ch, tots0):
                pltpu.sync_copy(
                    x_hbm.at[pl.ds(s * FPT + ch * CF, CF), pl.ds(468, 75)],
                    hbuf)

                def group(g, tots):
                    tL, tR = tots
                    fidx = g * L + lane

                    def cnt(lmbase):
                        def body(e, a):
                            lm = jnp.full(
                                (L,), lmbase + lax.shift_right_logical(e, 1),
                                jnp.int32)
                            cc = jnp.full(
                                (L,), jnp.bitwise_and(e, 1), jnp.int32)
                            v = plsc.load_gather(hbuf, [fidx, lm, cc])
                            return a + (v != v).astype(jnp.int32)

                        return lax.fori_loop(
                            0, 42, body, jnp.zeros((L,), jnp.int32))

                    aL = cnt(0)
                    aR = cnt(54)
                    cntL_l[pl.ds(ch * CF + g * L, L)] = aL
                    cntR_l[pl.ds(ch * CF + g * L, L)] = aR
                    return (tL + aL, tR + aR)

                return lax.fori_loop(0, CF // L, group, tots0)

            totL, totR = lax.fori_loop(
                0, FPT // CF, p1chunk,
                (jnp.zeros((L,), jnp.int32), jnp.zeros((L,), jnp.int32)),
            )
            stageL[...] = jnp.full((L,), jnp.sum(totL), jnp.int32)
            stageR[...] = jnp.full((L,), jnp.sum(totR), jnp.int32)
            pltpu.sync_copy(cntL_l, sh_cntL.at[pl.ds(s * FPT, FPT)])
            pltpu.sync_copy(cntR_l, sh_cntR.at[pl.ds(s * FPT, FPT)])
            pltpu.sync_copy(stageL, sh_totL.at[s])
            pltpu.sync_copy(stageR, sh_totR.at[s])
            pltpu.sync_copy(lidxL_hbm, lidxL_v)
            pltpu.sync_copy(lidxR_hbm, lidxR_v)
            pltpu.sync_copy(mulR_hbm, mulR_v)
            pltpu.sync_copy(addR_hbm, addR_v)
            plsc.subcore_barrier()

            @pl.when(s == 0)
            def _tile0():
                pltpu.sync_copy(sh_totL, totL_all)
                pltpu.sync_copy(sh_totR, totR_all)

                def tot_body(i, a):
                    aL, aR = a
                    return (aL + totL_all[i, :], aR + totR_all[i, :])

                accL, accR = lax.fori_loop(
                    0, NS, tot_body,
                    (jnp.zeros((L,), jnp.int32), jnp.zeros((L,), jnp.int32)),
                )
                ld = accL <= accR
                stageL[...] = ld.astype(jnp.int32)
                pltpu.sync_copy(stageL, sh_ld)

                pltpu.sync_copy(sh_cntL, cntL_all)
                pltpu.sync_copy(sh_cntR, cntR_all)

                def zero(i, carry):
                    idx_buf[pl.ds(i * L, L)] = jnp.zeros((L,), jnp.int32)
                    return carry

                lax.fori_loop(0, N_FRAMES // L, zero, 0)

                def comp(g, carry):
                    cl = cntL_all[pl.ds(g * L, L)]
                    cr = cntR_all[pl.ds(g * L, L)]
                    cnt = jnp.where(ld, cl, cr)
                    m = cnt < 42
                    mi = m.astype(jnp.int32)
                    pos = carry + plsc.cumsum(mi) - mi
                    fid = g * L + lane
                    plsc.store_scatter(idx_buf, [pos], fid, mask=m)
                    return carry + plsc.all_reduce_population_count(m)

                lax.fori_loop(
                    0, N_FRAMES // L, comp, jnp.zeros((L,), jnp.int32))
                pltpu.sync_copy(idx_buf, oidx_hbm)
                for v in range(OUT_F // L):
                    sl = pl.ds(v * L, L)
                    idx64[sl] = idx_buf[sl]
                pltpu.sync_copy(idx64, sh_idx)

            plsc.subcore_barrier()

            pltpu.sync_copy(sh_idx, idx64)
            pltpu.sync_copy(sh_ld, stageR)
            ldv = stageR[...] != 0

            for v in range(NGATHER):
                sl = pl.ds(v * L, L)
                lidx_sel[sl] = jnp.where(ldv, lidxL_v[sl], lidxR_v[sl])
                mul_sel[sl] = jnp.where(
                    ldv, jnp.full((L,), 1.0, jnp.float32), mulR_v[sl])
                add_sel[sl] = jnp.where(
                    ldv, jnp.full((L,), 0.0, jnp.float32), addR_v[sl])

            zero16 = jnp.zeros((L,), jnp.int32)
            for j in range(FPT3):
                fo = s * FPT3 + j
                ch_i = lax.div(fo, L)
                lane_i = lax.rem(fo, L)
                idxv = idx64[pl.ds(ch_i * L, L)]
                sel = jnp.sum(jnp.where(lane == lane_i, idxv, zero16))
                pltpu.sync_copy(x_hbm.at[pl.ds(sel, 1)], fbuf)
                z16 = jnp.full((L,), 0, jnp.int32)
                for v in range(NGATHER):
                    sl = pl.ds(v * L, L)
                    fl = lidx_sel[sl]
                    lm = lax.shift_right_logical(fl, 1)
                    cc = jnp.bitwise_and(fl, 1)
                    vals = plsc.load_gather(fbuf, [z16, lm, cc])
                    t = vals * mul_sel[sl] + add_sel[sl]
                    t = jnp.where(vals != vals, jnp.float32(0.0), t)
                    obuf[pl.ds((j * NGATHER + v) * L, L)] = t
            pltpu.sync_copy(
                obuf, out1_hbm.at[pl.ds(s * FPT3 * OUT_W, FPT3 * OUT_W)])

    return _sc_kernel


def kernel(x):
    out1, oidx = _build_sc_kernel()(
        x,
        jnp.asarray(_LIDX_L), jnp.asarray(_LIDX_R),
        jnp.asarray(_MUL_R), jnp.asarray(_ADD_R),
    )
    x1 = out1.reshape(OUT_F, OUT_W)[:, : 2 * N_OUT_LM].reshape(
        OUT_F, N_OUT_LM, 2)
    return (x1, oidx)


# speedup vs baseline: 53.7840x; 53.7840x over previous
import functools

import jax
import jax.numpy as jnp
import numpy as np
from jax import lax
from jax.experimental import pallas as pl
from jax.experimental.pallas import tpu as pltpu
from jax.experimental.pallas import tpu_sc as plsc

_LEFT_HAND = np.arange(468, 489)
_LEFT_POSE = np.array([502, 504, 506, 508, 510])
_LIPS = np.array([
    61, 185, 40, 39, 37, 0, 267, 269, 270, 409, 291, 146, 91, 181, 84, 17,
    314, 405, 321, 375, 78, 191, 80, 81, 82, 13, 312, 311, 310, 415, 95, 88,
    178, 87, 14, 317, 402, 318, 324, 308,
])
_RIGHT_HAND = np.arange(522, 543)
_RIGHT_POSE = np.array([503, 505, 507, 509, 511])

N_FRAMES = 4096
N_LM = 543
N_OUT_LM = 66
OUT_F = 64
NS = 16
L = 16
FPT = N_FRAMES // NS
NGATHER = 9
OUT_W = NGATHER * L
FPT3 = OUT_F // NS
NFT = N_FRAMES // 128
NROW = N_LM * 2 * NFT

_UNION_LM = np.concatenate(
    (_LIPS, _LEFT_HAND, _LEFT_POSE, _RIGHT_HAND, _RIGHT_POSE))

_HANDS = np.concatenate((_LEFT_HAND, _RIGHT_HAND))
_HTAB = np.zeros(192, np.int32)
for _h in range(42):
    for _q in range(4):
        _HTAB[_h * 4 + _q] = _HANDS[_h] * 64 + (_q & 1) + (_q >> 1) * 2
for _h in range(42):
    for _ft in range(2):
        for _c in range(2):
            _HTAB[_h * 4 + _ft * 2 + _c] = _HANDS[_h] * 64 + _ft * 2 + _c

_UB = np.zeros(192, np.int32)
for _u in range(92):
    for _c in range(2):
        _UB[2 * _u + _c] = _UNION_LM[_u] * 64 + _c

_LIDXU_L = np.pad(np.arange(132, dtype=np.int32), (0, OUT_W - 132))
_uR = np.concatenate((np.arange(40), np.arange(66, 92)))
_LIDXU_R = np.pad(
    np.stack([2 * _uR, 2 * _uR + 1], axis=1).reshape(-1).astype(np.int32),
    (0, OUT_W - 132))

_MUL_R = np.ones(OUT_W, np.float32)
_ADD_R = np.zeros(OUT_W, np.float32)
for _l in range(40, N_OUT_LM):
    _MUL_R[2 * _l] = -1.0
    _ADD_R[2 * _l] = 1.0


@functools.cache
def _build_sc_kernel():
    mesh = plsc.VectorSubcoreMesh(
        core_axis_name="c", subcore_axis_name="s", num_cores=2,
        num_subcores=NS,
    )

    @functools.partial(
        pl.kernel,
        out_type=[
            jax.ShapeDtypeStruct((OUT_F * OUT_W,), jnp.float32),
            jax.ShapeDtypeStruct((N_FRAMES,), jnp.int32),
        ],
        mesh=mesh,
        compiler_params=pltpu.CompilerParams(
            needs_layout_passes=False, use_tc_tiling_on_sc=False),
        scratch_types=[
            pltpu.VMEM((192, 128), jnp.float32),
            pltpu.VMEM((128,), jnp.int32),
            pltpu.VMEM((64,), jnp.int32),
            pltpu.VMEM((192,), jnp.int32),
            pltpu.VMEM((192,), jnp.int32),
            pltpu.VMEM((FPT,), jnp.int32),
            pltpu.VMEM((FPT,), jnp.int32),
            pltpu.VMEM((L,), jnp.int32),
            pltpu.VMEM((L,), jnp.int32),
            pltpu.VMEM_SHARED((N_FRAMES,), jnp.int32),
            pltpu.VMEM_SHARED((N_FRAMES,), jnp.int32),
            pltpu.VMEM_SHARED((NS, L), jnp.int32),
            pltpu.VMEM_SHARED((NS, L), jnp.int32),
            pltpu.VMEM_SHARED((OUT_F,), jnp.int32),
            pltpu.VMEM_SHARED((L,), jnp.int32),
            pltpu.VMEM((N_FRAMES,), jnp.int32),
            pltpu.VMEM((N_FRAMES,), jnp.int32),
            pltpu.VMEM((NS, L), jnp.int32),
            pltpu.VMEM((NS, L), jnp.int32),
            pltpu.VMEM((N_FRAMES,), jnp.int32),
            pltpu.VMEM((OUT_F,), jnp.int32),
            pltpu.VMEM((FPT3 * OUT_W,), jnp.float32),
            pltpu.VMEM((OUT_W,), jnp.int32),
            pltpu.VMEM((OUT_W,), jnp.int32),
            pltpu.VMEM((OUT_W,), jnp.float32),
            pltpu.VMEM((OUT_W,), jnp.float32),
            pltpu.VMEM((OUT_W,), jnp.int32),
            pltpu.VMEM((OUT_W,), jnp.float32),
            pltpu.VMEM((OUT_W,), jnp.float32),
            pltpu.SemaphoreType.DMA,
        ],
    )
    def _sc_kernel(
        y_hbm, htab_hbm, ub_hbm, lidxL_hbm, lidxR_hbm, mulR_hbm, addR_hbm,
        out1_hbm, oidx_hbm,
        gbuf, idxA, idxB, htab_v, ub_v,
        cntL_l, cntR_l, stageL, stageR,
        sh_cntL, sh_cntR, sh_totL, sh_totR, sh_idx, sh_ld,
        cntL_all, cntR_all, totL_all, totR_all,
        idx_buf, idx64, obuf,
        lidxL_v, lidxR_v, mulR_v, addR_v, lidx_sel, mul_sel, add_sel,
        sem,
    ):
        c = lax.axis_index("c")
        s = lax.axis_index("s")

        @pl.when(c == 0)
        def _core0():
            lane = lax.iota(jnp.int32, L)

            pltpu.sync_copy(htab_hbm, htab_v)
            s4 = s * 4
            for v in range(8):
                sl = pl.ds(v * L, L)
                idxA[sl] = htab_v[sl] + s4
            for v in range(4):
                idxB[pl.ds(v * L, L)] = htab_v[pl.ds(128 + v * L, L)] + s4
            pltpu.async_copy(y_hbm.at[idxA], gbuf.at[pl.ds(0, 128)], sem).wait()
            pltpu.async_copy(y_hbm.at[idxB], gbuf.at[pl.ds(128, 64)], sem).wait()

            totL = jnp.zeros((L,), jnp.int32)
            totR = jnp.zeros((L,), jnp.int32)
            for ft in range(2):
                for v in range(8):
                    sl = pl.ds(v * L, L)

                    def cnt(hbase, ft=ft, sl=sl):
                        def body(e, a):
                            row = ((hbase + lax.shift_right_logical(e, 1)) * 4
                                   + ft * 2 + jnp.bitwise_and(e, 1))
                            vv = gbuf[row, sl]
                            return a + (vv != vv).astype(jnp.int32)

                        return lax.fori_loop(
                            0, 42, body, jnp.zeros((L,), jnp.int32))

                    aL = cnt(0)
                    aR = cnt(21)
                    cntL_l[pl.ds(ft * 128 + v * L, L)] = aL
                    cntR_l[pl.ds(ft * 128 + v * L, L)] = aR
                    totL = totL + aL
                    totR = totR + aR
            stageL[...] = jnp.full((L,), jnp.sum(totL), jnp.int32)
            stageR[...] = jnp.full((L,), jnp.sum(totR), jnp.int32)
            pltpu.sync_copy(cntL_l, sh_cntL.at[pl.ds(s * FPT, FPT)])
            pltpu.sync_copy(cntR_l, sh_cntR.at[pl.ds(s * FPT, FPT)])
            pltpu.sync_copy(stageL, sh_totL.at[s])
            pltpu.sync_copy(stageR, sh_totR.at[s])
            pltpu.sync_copy(ub_hbm, ub_v)
            pltpu.sync_copy(lidxL_hbm, lidxL_v)
            pltpu.sync_copy(lidxR_hbm, lidxR_v)
            pltpu.sync_copy(mulR_hbm, mulR_v)
            pltpu.sync_copy(addR_hbm, addR_v)
            plsc.subcore_barrier()

            @pl.when(s == 0)
            def _tile0():
                pltpu.sync_copy(sh_totL, totL_all)
                pltpu.sync_copy(sh_totR, totR_all)

                def tot_body(i, a):
                    aL, aR = a
                    return (aL + totL_all[i, :], aR + totR_all[i, :])

                accL, accR = lax.fori_loop(
                    0, NS, tot_body,
                    (jnp.zeros((L,), jnp.int32), jnp.zeros((L,), jnp.int32)),
                )
                ld = accL <= accR
                stageL[...] = ld.astype(jnp.int32)
                pltpu.sync_copy(stageL, sh_ld)

                pltpu.sync_copy(sh_cntL, cntL_all)
                pltpu.sync_copy(sh_cntR, cntR_all)

                def zero(i, carry):
                    idx_buf[pl.ds(i * L, L)] = jnp.zeros((L,), jnp.int32)
                    return carry

                lax.fori_loop(0, N_FRAMES // L, zero, 0)

                def comp(g, carry):
                    cl = cntL_all[pl.ds(g * L, L)]
                    cr = cntR_all[pl.ds(g * L, L)]
                    cnt = jnp.where(ld, cl, cr)
                    m = cnt < 42
                    mi = m.astype(jnp.int32)
                    pos = carry + plsc.cumsum(mi) - mi
                    fid = g * L + lane
                    plsc.store_scatter(idx_buf, [pos], fid, mask=m)
                    return carry + plsc.all_reduce_population_count(m)

                lax.fori_loop(
                    0, N_FRAMES // L, comp, jnp.zeros((L,), jnp.int32))
                pltpu.sync_copy(idx_buf, oidx_hbm)
                for v in range(OUT_F // L):
                    sl = pl.ds(v * L, L)
                    idx64[sl] = idx_buf[sl]
                pltpu.sync_copy(idx64, sh_idx)

            plsc.subcore_barrier()

            pltpu.sync_copy(sh_idx, idx64)
            pltpu.sync_copy(sh_ld, stageR)
            ldv = stageR[...] != 0

            for v in range(NGATHER):
                sl = pl.ds(v * L, L)
                lidx_sel[sl] = jnp.where(ldv, lidxL_v[sl], lidxR_v[sl])
                mul_sel[sl] = jnp.where(
                    ldv, jnp.full((L,), 1.0, jnp.float32), mulR_v[sl])
                add_sel[sl] = jnp.where(
                    ldv, jnp.full((L,), 0.0, jnp.float32), addR_v[sl])

            zero16 = jnp.zeros((L,), jnp.int32)
            for j in range(FPT3):
                fo = s * FPT3 + j
                ch_i = lax.div(fo, L)
                lane_i = lax.rem(fo, L)
                idxv = idx64[pl.ds(ch_i * L, L)]
                sel = jnp.sum(jnp.where(lane == lane_i, idxv, zero16))
                ft2 = lax.shift_right_logical(sel, 7) * 2
                fl = jnp.bitwise_and(sel, 127)
                for v in range(8):
                    sl = pl.ds(v * L, L)
                    idxA[sl] = ub_v[sl] + ft2
                for v in range(4):
                    idxB[pl.ds(v * L, L)] = ub_v[pl.ds(128 + v * L, L)] + ft2
                pltpu.async_copy(
                    y_hbm.at[idxA], gbuf.at[pl.ds(0, 128)], sem).wait()
                pltpu.async_copy(
                    y_hbm.at[idxB], gbuf.at[pl.ds(128, 64)], sem).wait()
                flv = jnp.full((L,), fl, jnp.int32)
                for v in range(NGATHER):
                    sl = pl.ds(v * L, L)
                    vals = plsc.load_gather(gbuf, [lidx_sel[sl], flv])
                    t = vals * mul_sel[sl] + add_sel[sl]
                    t = jnp.where(vals != vals, jnp.float32(0.0), t)
                    obuf[pl.ds((j * NGATHER + v) * L, L)] = t
            pltpu.sync_copy(
                obuf, out1_hbm.at[pl.ds(s * FPT3 * OUT_W, FPT3 * OUT_W)])

    return _sc_kernel


def kernel(x):
    y = x.transpose(1, 0, 2).reshape(N_LM, NFT, 128, 2)
    y = y.transpose(0, 1, 3, 2).reshape(NROW, 128)
    out1, oidx = _build_sc_kernel()(
        y,
        jnp.asarray(_HTAB), jnp.asarray(_UB),
        jnp.asarray(_LIDXU_L), jnp.asarray(_LIDXU_R),
        jnp.asarray(_MUL_R), jnp.asarray(_ADD_R),
    )
    x1 = out1.reshape(OUT_F, OUT_W)[:, : 2 * N_OUT_LM].reshape(
        OUT_F, N_OUT_LM, 2)
    return (x1, oidx)


# speedup vs baseline: 54.7895x; 1.0187x over previous
import functools

import jax
import jax.numpy as jnp
import numpy as np
from jax import lax
from jax.experimental import pallas as pl
from jax.experimental.pallas import tpu as pltpu
from jax.experimental.pallas import tpu_sc as plsc

_LEFT_HAND = np.arange(468, 489)
_LEFT_POSE = np.array([502, 504, 506, 508, 510])
_LIPS = np.array([
    61, 185, 40, 39, 37, 0, 267, 269, 270, 409, 291, 146, 91, 181, 84, 17,
    314, 405, 321, 375, 78, 191, 80, 81, 82, 13, 312, 311, 310, 415, 95, 88,
    178, 87, 14, 317, 402, 318, 324, 308,
])
_RIGHT_HAND = np.arange(522, 543)
_RIGHT_POSE = np.array([503, 505, 507, 509, 511])

N_FRAMES = 4096
N_LM = 543
N_OUT_LM = 66
OUT_F = 64
NS = 16
L = 16
FPT = N_FRAMES // NS
NGATHER = 9
OUT_W = NGATHER * L
FPT3 = OUT_F // NS
NFT = N_FRAMES // 128
NROW = N_LM * 2 * NFT

_UNION_LM = np.concatenate(
    (_LIPS, _LEFT_HAND, _LEFT_POSE, _RIGHT_HAND, _RIGHT_POSE))

_HANDS = np.concatenate((_LEFT_HAND, _RIGHT_HAND))
_HTAB = np.zeros(192, np.int32)
for _h in range(42):
    for _q in range(4):
        _HTAB[_h * 4 + _q] = _HANDS[_h] * 64 + (_q & 1) + (_q >> 1) * 2
for _h in range(42):
    for _ft in range(2):
        for _c in range(2):
            _HTAB[_h * 4 + _ft * 2 + _c] = _HANDS[_h] * 64 + _ft * 2 + _c

_UB = np.zeros(192, np.int32)
for _u in range(92):
    for _c in range(2):
        _UB[2 * _u + _c] = _UNION_LM[_u] * 64 + _c

_LIDXU_L = np.pad(np.arange(132, dtype=np.int32), (0, OUT_W - 132))
_uR = np.concatenate((np.arange(40), np.arange(66, 92)))
_LIDXU_R = np.pad(
    np.stack([2 * _uR, 2 * _uR + 1], axis=1).reshape(-1).astype(np.int32),
    (0, OUT_W - 132))

_MUL_R = np.ones(OUT_W, np.float32)
_ADD_R = np.zeros(OUT_W, np.float32)
for _l in range(40, N_OUT_LM):
    _MUL_R[2 * _l] = -1.0
    _ADD_R[2 * _l] = 1.0


@functools.cache
def _build_sc_kernel():
    mesh = plsc.VectorSubcoreMesh(
        core_axis_name="c", subcore_axis_name="s", num_cores=2,
        num_subcores=NS,
    )

    @functools.partial(
        pl.kernel,
        out_type=[
            jax.ShapeDtypeStruct((OUT_F * OUT_W,), jnp.float32),
            jax.ShapeDtypeStruct((N_FRAMES,), jnp.int32),
        ],
        mesh=mesh,
        compiler_params=pltpu.CompilerParams(
            needs_layout_passes=False, use_tc_tiling_on_sc=False),
        scratch_types=[
            pltpu.VMEM((192, 128), jnp.float32),
            pltpu.VMEM((128,), jnp.int32),
            pltpu.VMEM((64,), jnp.int32),
            pltpu.VMEM((192,), jnp.int32),
            pltpu.VMEM((192,), jnp.int32),
            pltpu.VMEM((FPT,), jnp.int32),
            pltpu.VMEM((FPT,), jnp.int32),
            pltpu.VMEM((L,), jnp.int32),
            pltpu.VMEM((L,), jnp.int32),
            pltpu.VMEM_SHARED((N_FRAMES,), jnp.int32),
            pltpu.VMEM_SHARED((N_FRAMES,), jnp.int32),
            pltpu.VMEM_SHARED((NS, L), jnp.int32),
            pltpu.VMEM_SHARED((NS, L), jnp.int32),
            pltpu.VMEM_SHARED((OUT_F,), jnp.int32),
            pltpu.VMEM((N_FRAMES,), jnp.int32),
            pltpu.VMEM((N_FRAMES,), jnp.int32),
            pltpu.VMEM((NS, L), jnp.int32),
            pltpu.VMEM((NS, L), jnp.int32),
            pltpu.VMEM((N_FRAMES,), jnp.int32),
            pltpu.VMEM((OUT_F,), jnp.int32),
            pltpu.VMEM((FPT3 * OUT_W,), jnp.float32),
            pltpu.VMEM((OUT_W,), jnp.int32),
            pltpu.VMEM((OUT_W,), jnp.int32),
            pltpu.VMEM((OUT_W,), jnp.float32),
            pltpu.VMEM((OUT_W,), jnp.float32),
            pltpu.VMEM((OUT_W,), jnp.int32),
            pltpu.VMEM((OUT_W,), jnp.float32),
            pltpu.VMEM((OUT_W,), jnp.float32),
            pltpu.SemaphoreType.DMA,
        ],
    )
    def _sc_kernel(
        y_hbm, htab_hbm, ub_hbm, lidxL_hbm, lidxR_hbm, mulR_hbm, addR_hbm,
        out1_hbm, oidx_hbm,
        gbuf, idxA, idxB, htab_v, ub_v,
        cntL_l, cntR_l, stageL, stageR,
        sh_cntL, sh_cntR, sh_totL, sh_totR, sh_idx,
        cntL_all, cntR_all, totL_all, totR_all,
        idx_buf, idx64, obuf,
        lidxL_v, lidxR_v, mulR_v, addR_v, lidx_sel, mul_sel, add_sel,
        sem,
    ):
        c = lax.axis_index("c")
        s = lax.axis_index("s")

        @pl.when(c == 0)
        def _core0():
            lane = lax.iota(jnp.int32, L)

            pltpu.sync_copy(htab_hbm, htab_v)
            s4 = s * 4
            for v in range(8):
                sl = pl.ds(v * L, L)
                idxA[sl] = htab_v[sl] + s4
            for v in range(4):
                idxB[pl.ds(v * L, L)] = htab_v[pl.ds(128 + v * L, L)] + s4
            pltpu.async_copy(y_hbm.at[idxA], gbuf.at[pl.ds(0, 128)], sem).wait()
            pltpu.async_copy(y_hbm.at[idxB], gbuf.at[pl.ds(128, 64)], sem).wait()

            totL = jnp.zeros((L,), jnp.int32)
            totR = jnp.zeros((L,), jnp.int32)
            for ft in range(2):
                for v in range(8):
                    sl = pl.ds(v * L, L)
                    ft2 = ft * 2

                    def body(h, a, ft2=ft2, sl=sl):
                        aL, aR = a
                        r = h * 4 + ft2
                        v0 = gbuf[r, sl]
                        v1 = gbuf[r + 1, sl]
                        v2 = gbuf[r + 84, sl]
                        v3 = gbuf[r + 85, sl]
                        aL = (aL + (v0 != v0).astype(jnp.int32)
                              + (v1 != v1).astype(jnp.int32))
                        aR = (aR + (v2 != v2).astype(jnp.int32)
                              + (v3 != v3).astype(jnp.int32))
                        return (aL, aR)

                    aL, aR = lax.fori_loop(
                        0, 21, body,
                        (jnp.zeros((L,), jnp.int32),
                         jnp.zeros((L,), jnp.int32)))
                    cntL_l[pl.ds(ft * 128 + v * L, L)] = aL
                    cntR_l[pl.ds(ft * 128 + v * L, L)] = aR
                    totL = totL + aL
                    totR = totR + aR
            stageL[...] = jnp.full((L,), jnp.sum(totL), jnp.int32)
            stageR[...] = jnp.full((L,), jnp.sum(totR), jnp.int32)
            pltpu.sync_copy(cntL_l, sh_cntL.at[pl.ds(s * FPT, FPT)])
            pltpu.sync_copy(cntR_l, sh_cntR.at[pl.ds(s * FPT, FPT)])
            pltpu.sync_copy(stageL, sh_totL.at[s])
            pltpu.sync_copy(stageR, sh_totR.at[s])
            pltpu.sync_copy(ub_hbm, ub_v)
            pltpu.sync_copy(lidxL_hbm, lidxL_v)
            pltpu.sync_copy(lidxR_hbm, lidxR_v)
            pltpu.sync_copy(mulR_hbm, mulR_v)
            pltpu.sync_copy(addR_hbm, addR_v)
            plsc.subcore_barrier()

            pltpu.sync_copy(sh_totL, totL_all)
            pltpu.sync_copy(sh_totR, totR_all)

            def tot_body(i, a):
                aL, aR = a
                return (aL + totL_all[i, :], aR + totR_all[i, :])

            accL, accR = lax.fori_loop(
                0, NS, tot_body,
                (jnp.zeros((L,), jnp.int32), jnp.zeros((L,), jnp.int32)),
            )
            ldv = accL <= accR

            @pl.when(s == 0)
            def _tile0():
                pltpu.sync_copy(sh_cntL, cntL_all)
                pltpu.sync_copy(sh_cntR, cntR_all)

                def comp(g, carry):
                    cl = cntL_all[pl.ds(g * L, L)]
                    cr = cntR_all[pl.ds(g * L, L)]
                    cnt = jnp.where(ldv, cl, cr)
                    m = cnt < 42
                    mi = m.astype(jnp.int32)
                    pos = carry + plsc.cumsum(mi) - mi
                    fid = g * L + lane
                    plsc.store_scatter(idx_buf, [pos], fid, mask=m)
                    return carry + plsc.all_reduce_population_count(m)

                count = lax.fori_loop(
                    0, N_FRAMES // L, comp, jnp.zeros((L,), jnp.int32))

                cnt0 = jnp.sum(jnp.where(lane == 0, count, 0))
                g0 = lax.shift_right_logical(cnt0, 4)
                pad = lane >= jnp.bitwise_and(cnt0, 15)
                plsc.store_scatter(
                    idx_buf, [g0 * L + lane],
                    jnp.zeros((L,), jnp.int32),
                    mask=jnp.logical_and(pad, g0 * L + lane < N_FRAMES))

                def zero(i, carry):
                    idx_buf[pl.ds(i * L, L)] = jnp.zeros((L,), jnp.int32)
                    return carry

                lax.fori_loop(g0 + 1, N_FRAMES // L, zero, 0)
                pltpu.sync_copy(idx_buf, oidx_hbm)
                for v in range(OUT_F // L):
                    sl = pl.ds(v * L, L)
                    idx64[sl] = idx_buf[sl]
                pltpu.sync_copy(idx64, sh_idx)

            for v in range(NGATHER):
                sl = pl.ds(v * L, L)
                lidx_sel[sl] = jnp.where(ldv, lidxL_v[sl], lidxR_v[sl])
                mul_sel[sl] = jnp.where(
                    ldv, jnp.full((L,), 1.0, jnp.float32), mulR_v[sl])
                add_sel[sl] = jnp.where(
                    ldv, jnp.full((L,), 0.0, jnp.float32), addR_v[sl])

            plsc.subcore_barrier()

            pltpu.sync_copy(sh_idx, idx64)
            zero16 = jnp.zeros((L,), jnp.int32)
            for j in range(FPT3):
                fo = s * FPT3 + j
                ch_i = lax.div(fo, L)
                lane_i = lax.rem(fo, L)
                idxv = idx64[pl.ds(ch_i * L, L)]
                sel = jnp.sum(jnp.where(lane == lane_i, idxv, zero16))
                ft2 = lax.shift_right_logical(sel, 7) * 2
                fl = jnp.bitwise_and(sel, 127)
                for v in range(8):
                    sl = pl.ds(v * L, L)
                    idxA[sl] = ub_v[sl] + ft2
                for v in range(4):
                    idxB[pl.ds(v * L, L)] = ub_v[pl.ds(128 + v * L, L)] + ft2
                pltpu.async_copy(
                    y_hbm.at[idxA], gbuf.at[pl.ds(0, 128)], sem).wait()
                pltpu.async_copy(
                    y_hbm.at[idxB], gbuf.at[pl.ds(128, 64)], sem).wait()
                flv = jnp.full((L,), fl, jnp.int32)
                for v in range(NGATHER):
                    sl = pl.ds(v * L, L)
                    vals = plsc.load_gather(gbuf, [lidx_sel[sl], flv])
                    t = vals * mul_sel[sl] + add_sel[sl]
                    t = jnp.where(vals != vals, jnp.float32(0.0), t)
                    obuf[pl.ds((j * NGATHER + v) * L, L)] = t
            pltpu.sync_copy(
                obuf, out1_hbm.at[pl.ds(s * FPT3 * OUT_W, FPT3 * OUT_W)])

    return _sc_kernel


def kernel(x):
    y = x.transpose(1, 0, 2).reshape(N_LM, NFT, 128, 2)
    y = y.transpose(0, 1, 3, 2).reshape(NROW, 128)
    out1, oidx = _build_sc_kernel()(
        y,
        jnp.asarray(_HTAB), jnp.asarray(_UB),
        jnp.asarray(_LIDXU_L), jnp.asarray(_LIDXU_R),
        jnp.asarray(_MUL_R), jnp.asarray(_ADD_R),
    )
    x1 = out1.reshape(OUT_F, OUT_W)[:, : 2 * N_OUT_LM].reshape(
        OUT_F, N_OUT_LM, 2)
    return (x1, oidx)


# speedup vs baseline: 59.9333x; 1.0939x over previous
import functools

import jax
import jax.numpy as jnp
import numpy as np
from jax import lax
from jax.experimental import pallas as pl
from jax.experimental.pallas import tpu as pltpu
from jax.experimental.pallas import tpu_sc as plsc

_LEFT_HAND = np.arange(468, 489)
_LEFT_POSE = np.array([502, 504, 506, 508, 510])
_LIPS = np.array([
    61, 185, 40, 39, 37, 0, 267, 269, 270, 409, 291, 146, 91, 181, 84, 17,
    314, 405, 321, 375, 78, 191, 80, 81, 82, 13, 312, 311, 310, 415, 95, 88,
    178, 87, 14, 317, 402, 318, 324, 308,
])
_RIGHT_HAND = np.arange(522, 543)
_RIGHT_POSE = np.array([503, 505, 507, 509, 511])

N_FRAMES = 4096
N_LM = 543
N_OUT_LM = 66
OUT_F = 64
NS = 16
L = 16
FPT = N_FRAMES // NS
NGATHER = 9
OUT_W = NGATHER * L
FPT3 = OUT_F // NS
NFT = N_FRAMES // 128
NROW = N_LM * 2 * NFT

_UNION_LM = np.concatenate(
    (_LIPS, _LEFT_HAND, _LEFT_POSE, _RIGHT_HAND, _RIGHT_POSE))

_HANDS = np.concatenate((_LEFT_HAND, _RIGHT_HAND))
_HTAB = np.zeros(192, np.int32)
for _h in range(42):
    for _q in range(4):
        _HTAB[_h * 4 + _q] = _HANDS[_h] * 64 + (_q & 1) + (_q >> 1) * 2
for _h in range(42):
    for _ft in range(2):
        for _c in range(2):
            _HTAB[_h * 4 + _ft * 2 + _c] = _HANDS[_h] * 64 + _ft * 2 + _c

_UB = np.zeros(192, np.int32)
for _u in range(92):
    for _c in range(2):
        _UB[2 * _u + _c] = _UNION_LM[_u] * 64 + _c

_LIDXU_L = np.pad(np.arange(132, dtype=np.int32), (0, OUT_W - 132))
_uR = np.concatenate((np.arange(40), np.arange(66, 92)))
_LIDXU_R = np.pad(
    np.stack([2 * _uR, 2 * _uR + 1], axis=1).reshape(-1).astype(np.int32),
    (0, OUT_W - 132))

_MUL_R = np.ones(OUT_W, np.float32)
_ADD_R = np.zeros(OUT_W, np.float32)
for _l in range(40, N_OUT_LM):
    _MUL_R[2 * _l] = -1.0
    _ADD_R[2 * _l] = 1.0

_TBL = np.concatenate([
    _HTAB, _UB, _LIDXU_L, _LIDXU_R,
    _MUL_R.view(np.int32), _ADD_R.view(np.int32),
]).astype(np.int32)


@functools.cache
def _build_sc_kernel():
    mesh = plsc.VectorSubcoreMesh(
        core_axis_name="c", subcore_axis_name="s", num_cores=2,
        num_subcores=NS,
    )

    @functools.partial(
        pl.kernel,
        out_type=[
            jax.ShapeDtypeStruct((OUT_F * OUT_W,), jnp.float32),
            jax.ShapeDtypeStruct((N_FRAMES,), jnp.int32),
        ],
        mesh=mesh,
        compiler_params=pltpu.CompilerParams(
            needs_layout_passes=False, use_tc_tiling_on_sc=False),
        scratch_types=[
            pltpu.VMEM((192, 128), jnp.float32),
            pltpu.VMEM((128,), jnp.int32),
            pltpu.VMEM((64,), jnp.int32),
            pltpu.VMEM((960,), jnp.int32),
            pltpu.VMEM((FPT,), jnp.int32),
            pltpu.VMEM((FPT,), jnp.int32),
            pltpu.VMEM((L,), jnp.int32),
            pltpu.VMEM((L,), jnp.int32),
            pltpu.VMEM_SHARED((N_FRAMES,), jnp.int32),
            pltpu.VMEM_SHARED((N_FRAMES,), jnp.int32),
            pltpu.VMEM_SHARED((NS, L), jnp.int32),
            pltpu.VMEM_SHARED((NS, L), jnp.int32),
            pltpu.VMEM_SHARED((OUT_F,), jnp.int32),
            pltpu.VMEM((N_FRAMES,), jnp.int32),
            pltpu.VMEM((N_FRAMES,), jnp.int32),
            pltpu.VMEM((NS, L), jnp.int32),
            pltpu.VMEM((NS, L), jnp.int32),
            pltpu.VMEM((N_FRAMES,), jnp.int32),
            pltpu.VMEM((OUT_F,), jnp.int32),
            pltpu.VMEM((FPT3 * OUT_W,), jnp.float32),
            pltpu.VMEM((OUT_W,), jnp.int32),
            pltpu.VMEM((OUT_W,), jnp.float32),
            pltpu.VMEM((OUT_W,), jnp.float32),
            pltpu.SemaphoreType.DMA,
        ],
    )
    def _sc_kernel(
        y_hbm, tbl_hbm,
        out1_hbm, oidx_hbm,
        gbuf, idxA, idxB, tbl_v,
        cntL_l, cntR_l, stageL, stageR,
        sh_cntL, sh_cntR, sh_totL, sh_totR, sh_idx,
        cntL_all, cntR_all, totL_all, totR_all,
        idx_buf, idx64, obuf,
        lidx_sel, mul_sel, add_sel,
        sem,
    ):
        c = lax.axis_index("c")
        s = lax.axis_index("s")

        @pl.when(c == 0)
        def _core0():
            lane = lax.iota(jnp.int32, L)

            pltpu.sync_copy(tbl_hbm, tbl_v)
            s4 = s * 4
            for v in range(8):
                sl = pl.ds(v * L, L)
                idxA[sl] = tbl_v[sl] + s4
            for v in range(4):
                idxB[pl.ds(v * L, L)] = tbl_v[pl.ds(128 + v * L, L)] + s4
            pltpu.async_copy(y_hbm.at[idxA], gbuf.at[pl.ds(0, 128)], sem).wait()
            pltpu.async_copy(y_hbm.at[idxB], gbuf.at[pl.ds(128, 64)], sem).wait()

            totL = jnp.zeros((L,), jnp.int32)
            totR = jnp.zeros((L,), jnp.int32)
            for ft in range(2):
                for v in range(8):
                    sl = pl.ds(v * L, L)
                    ft2 = ft * 2

                    def body(h, a, ft2=ft2, sl=sl):
                        aL, aR = a
                        r = h * 4 + ft2
                        v0 = gbuf[r, sl]
                        v1 = gbuf[r + 1, sl]
                        v2 = gbuf[r + 84, sl]
                        v3 = gbuf[r + 85, sl]
                        aL = (aL + (v0 != v0).astype(jnp.int32)
                              + (v1 != v1).astype(jnp.int32))
                        aR = (aR + (v2 != v2).astype(jnp.int32)
                              + (v3 != v3).astype(jnp.int32))
                        return (aL, aR)

                    aL, aR = lax.fori_loop(
                        0, 21, body,
                        (jnp.zeros((L,), jnp.int32),
                         jnp.zeros((L,), jnp.int32)))
                    cntL_l[pl.ds(ft * 128 + v * L, L)] = aL
                    cntR_l[pl.ds(ft * 128 + v * L, L)] = aR
                    totL = totL + aL
                    totR = totR + aR
            stageL[...] = jnp.full((L,), jnp.sum(totL), jnp.int32)
            stageR[...] = jnp.full((L,), jnp.sum(totR), jnp.int32)
            pltpu.sync_copy(cntL_l, sh_cntL.at[pl.ds(s * FPT, FPT)])
            pltpu.sync_copy(cntR_l, sh_cntR.at[pl.ds(s * FPT, FPT)])
            pltpu.sync_copy(stageL, sh_totL.at[s])
            pltpu.sync_copy(stageR, sh_totR.at[s])
            plsc.subcore_barrier()

            pltpu.sync_copy(sh_totL, totL_all)
            pltpu.sync_copy(sh_totR, totR_all)

            def tot_body(i, a):
                aL, aR = a
                return (aL + totL_all[i, :], aR + totR_all[i, :])

            accL, accR = lax.fori_loop(
                0, NS, tot_body,
                (jnp.zeros((L,), jnp.int32), jnp.zeros((L,), jnp.int32)),
            )
            ldv = accL <= accR

            @pl.when(s == 0)
            def _tile0():
                pltpu.sync_copy(sh_cntL, cntL_all)
                pltpu.sync_copy(sh_cntR, cntR_all)

                def comp(g, carry):
                    cl = cntL_all[pl.ds(g * L, L)]
                    cr = cntR_all[pl.ds(g * L, L)]
                    cnt = jnp.where(ldv, cl, cr)
                    m = cnt < 42
                    mi = m.astype(jnp.int32)
                    pos = carry + plsc.cumsum(mi) - mi
                    fid = g * L + lane
                    plsc.store_scatter(idx_buf, [pos], fid, mask=m)
                    return carry + plsc.all_reduce_population_count(m)

                count = lax.fori_loop(
                    0, N_FRAMES // L, comp, jnp.zeros((L,), jnp.int32))

                cnt0 = jnp.sum(jnp.where(lane == 0, count, 0))
                g0 = lax.shift_right_logical(cnt0, 4)
                pad = lane >= jnp.bitwise_and(cnt0, 15)
                plsc.store_scatter(
                    idx_buf, [g0 * L + lane],
                    jnp.zeros((L,), jnp.int32),
                    mask=jnp.logical_and(pad, g0 * L + lane < N_FRAMES))

                def zero(i, carry):
                    idx_buf[pl.ds(i * L, L)] = jnp.zeros((L,), jnp.int32)
                    return carry

                lax.fori_loop(g0 + 1, N_FRAMES // L, zero, 0)
                pltpu.sync_copy(idx_buf, oidx_hbm)
                for v in range(OUT_F // L):
                    sl = pl.ds(v * L, L)
                    idx64[sl] = idx_buf[sl]
                pltpu.sync_copy(idx64, sh_idx)

            for v in range(NGATHER):
                sl = pl.ds(v * L, L)
                lidx_sel[sl] = jnp.where(
                    ldv, tbl_v[pl.ds(384 + v * L, L)],
                    tbl_v[pl.ds(528 + v * L, L)])
                mul_sel[sl] = jnp.where(
                    ldv, jnp.full((L,), 1.0, jnp.float32),
                    plsc.bitcast(tbl_v[pl.ds(672 + v * L, L)], jnp.float32))
                add_sel[sl] = jnp.where(
                    ldv, jnp.full((L,), 0.0, jnp.float32),
                    plsc.bitcast(tbl_v[pl.ds(816 + v * L, L)], jnp.float32))

            plsc.subcore_barrier()

            pltpu.sync_copy(sh_idx, idx64)
            zero16 = jnp.zeros((L,), jnp.int32)
            for j in range(FPT3):
                fo = s * FPT3 + j
                ch_i = lax.div(fo, L)
                lane_i = lax.rem(fo, L)
                idxv = idx64[pl.ds(ch_i * L, L)]
                sel = jnp.sum(jnp.where(lane == lane_i, idxv, zero16))
                ft2 = lax.shift_right_logical(sel, 7) * 2
                fl = jnp.bitwise_and(sel, 127)
                for v in range(8):
                    sl = pl.ds(v * L, L)
                    idxA[sl] = tbl_v[pl.ds(192 + v * L, L)] + ft2
                for v in range(4):
                    idxB[pl.ds(v * L, L)] = tbl_v[pl.ds(320 + v * L, L)] + ft2
                pltpu.async_copy(
                    y_hbm.at[idxA], gbuf.at[pl.ds(0, 128)], sem).wait()
                pltpu.async_copy(
                    y_hbm.at[idxB], gbuf.at[pl.ds(128, 64)], sem).wait()
                flv = jnp.full((L,), fl, jnp.int32)
                for v in range(NGATHER):
                    sl = pl.ds(v * L, L)
                    vals = plsc.load_gather(gbuf, [lidx_sel[sl], flv])
                    t = vals * mul_sel[sl] + add_sel[sl]
                    t = jnp.where(vals != vals, jnp.float32(0.0), t)
                    obuf[pl.ds((j * NGATHER + v) * L, L)] = t
            pltpu.sync_copy(
                obuf, out1_hbm.at[pl.ds(s * FPT3 * OUT_W, FPT3 * OUT_W)])

    return _sc_kernel


def kernel(x):
    y = x.transpose(1, 0, 2).reshape(N_LM, NFT, 128, 2)
    y = y.transpose(0, 1, 3, 2).reshape(NROW, 128)
    out1, oidx = _build_sc_kernel()(y, jnp.asarray(_TBL))
    x1 = out1.reshape(OUT_F, OUT_W)[:, : 2 * N_OUT_LM].reshape(
        OUT_F, N_OUT_LM, 2)
    return (x1, oidx)


# speedup vs baseline: 61.0123x; 1.0180x over previous
import functools

import jax
import jax.numpy as jnp
import numpy as np
from jax import lax
from jax.experimental import pallas as pl
from jax.experimental.pallas import tpu as pltpu
from jax.experimental.pallas import tpu_sc as plsc

_LEFT_HAND = np.arange(468, 489)
_LEFT_POSE = np.array([502, 504, 506, 508, 510])
_LIPS = np.array([
    61, 185, 40, 39, 37, 0, 267, 269, 270, 409, 291, 146, 91, 181, 84, 17,
    314, 405, 321, 375, 78, 191, 80, 81, 82, 13, 312, 311, 310, 415, 95, 88,
    178, 87, 14, 317, 402, 318, 324, 308,
])
_RIGHT_HAND = np.arange(522, 543)
_RIGHT_POSE = np.array([503, 505, 507, 509, 511])

N_FRAMES = 4096
N_LM = 543
N_OUT_LM = 66
OUT_F = 64
NS = 16
L = 16
FPT = N_FRAMES // NS
NGATHER = 9
OUT_W = NGATHER * L
FPT3 = OUT_F // NS
NFT = N_FRAMES // 128
NROW = N_LM * 2 * NFT

_UNION_LM = np.concatenate(
    (_LIPS, _LEFT_HAND, _LEFT_POSE, _RIGHT_HAND, _RIGHT_POSE))

_HANDS = np.concatenate((_LEFT_HAND, _RIGHT_HAND))
_HTAB = np.zeros(192, np.int32)
for _h in range(42):
    for _q in range(4):
        _HTAB[_h * 4 + _q] = _HANDS[_h] * 64 + (_q & 1) + (_q >> 1) * 2
for _h in range(42):
    for _ft in range(2):
        for _c in range(2):
            _HTAB[_h * 4 + _ft * 2 + _c] = _HANDS[_h] * 64 + _ft * 2 + _c

_UB = np.zeros(192, np.int32)
for _u in range(92):
    for _c in range(2):
        _UB[2 * _u + _c] = _UNION_LM[_u] * 64 + _c

_LIDXU_L = np.pad(np.arange(132, dtype=np.int32), (0, OUT_W - 132))
_uR = np.concatenate((np.arange(40), np.arange(66, 92)))
_LIDXU_R = np.pad(
    np.stack([2 * _uR, 2 * _uR + 1], axis=1).reshape(-1).astype(np.int32),
    (0, OUT_W - 132))

_MUL_R = np.ones(OUT_W, np.float32)
_ADD_R = np.zeros(OUT_W, np.float32)
for _l in range(40, N_OUT_LM):
    _MUL_R[2 * _l] = -1.0
    _ADD_R[2 * _l] = 1.0

_TBL = np.concatenate([
    _HTAB, _UB, _LIDXU_L, _LIDXU_R,
    _MUL_R.view(np.int32), _ADD_R.view(np.int32),
]).astype(np.int32)


@functools.cache
def _build_sc_kernel():
    mesh = plsc.VectorSubcoreMesh(
        core_axis_name="c", subcore_axis_name="s", num_cores=2,
        num_subcores=NS,
    )

    @functools.partial(
        pl.kernel,
        out_type=[
            jax.ShapeDtypeStruct((OUT_F * OUT_W,), jnp.float32),
            jax.ShapeDtypeStruct((N_FRAMES,), jnp.int32),
        ],
        mesh=mesh,
        compiler_params=pltpu.CompilerParams(
            needs_layout_passes=False, use_tc_tiling_on_sc=False),
        scratch_types=[
            pltpu.VMEM((2, 192, 128), jnp.float32),
            pltpu.VMEM((2, 128), jnp.int32),
            pltpu.VMEM((2, 64), jnp.int32),
            pltpu.VMEM((960,), jnp.int32),
            pltpu.VMEM((FPT,), jnp.int32),
            pltpu.VMEM((FPT,), jnp.int32),
            pltpu.VMEM((L,), jnp.int32),
            pltpu.VMEM((L,), jnp.int32),
            pltpu.VMEM_SHARED((N_FRAMES,), jnp.int32),
            pltpu.VMEM_SHARED((N_FRAMES,), jnp.int32),
            pltpu.VMEM_SHARED((NS, L), jnp.int32),
            pltpu.VMEM_SHARED((NS, L), jnp.int32),
            pltpu.VMEM_SHARED((OUT_F,), jnp.int32),
            pltpu.VMEM((N_FRAMES,), jnp.int32),
            pltpu.VMEM((N_FRAMES,), jnp.int32),
            pltpu.VMEM((NS, L), jnp.int32),
            pltpu.VMEM((NS, L), jnp.int32),
            pltpu.VMEM((N_FRAMES,), jnp.int32),
            pltpu.VMEM((OUT_F,), jnp.int32),
            pltpu.VMEM((FPT3 * OUT_W,), jnp.float32),
            pltpu.VMEM((OUT_W,), jnp.int32),
            pltpu.VMEM((OUT_W,), jnp.float32),
            pltpu.VMEM((OUT_W,), jnp.float32),
            pltpu.SemaphoreType.DMA,
            pltpu.SemaphoreType.DMA,
        ],
    )
    def _sc_kernel(
        y_hbm, tbl_hbm,
        out1_hbm, oidx_hbm,
        gbuf2, idxA2, idxB2, tbl_v,
        cntL_l, cntR_l, stageL, stageR,
        sh_cntL, sh_cntR, sh_totL, sh_totR, sh_idx,
        cntL_all, cntR_all, totL_all, totR_all,
        idx_buf, idx64, obuf,
        lidx_sel, mul_sel, add_sel,
        sem0, sem1,
    ):
        sems = (sem0, sem1)
        c = lax.axis_index("c")
        s = lax.axis_index("s")

        @pl.when(c == 0)
        def _core0():
            lane = lax.iota(jnp.int32, L)

            pltpu.sync_copy(tbl_hbm, tbl_v)
            s4 = s * 4
            for v in range(8):
                idxA2[0, pl.ds(v * L, L)] = tbl_v[pl.ds(v * L, L)] + s4
            for v in range(4):
                idxB2[0, pl.ds(v * L, L)] = tbl_v[pl.ds(128 + v * L, L)] + s4
            h1 = pltpu.async_copy(
                y_hbm.at[idxA2.at[0]], gbuf2.at[0, pl.ds(0, 128)], sem0)
            h2 = pltpu.async_copy(
                y_hbm.at[idxB2.at[0]], gbuf2.at[0, pl.ds(128, 64)], sem0)
            h1.wait()
            h2.wait()

            totL = jnp.zeros((L,), jnp.int32)
            totR = jnp.zeros((L,), jnp.int32)
            for ft in range(2):
                for v in range(8):
                    sl = pl.ds(v * L, L)
                    ft2 = ft * 2

                    def body(h, a, ft2=ft2, sl=sl):
                        aL, aR = a
                        r = h * 4 + ft2
                        v0 = gbuf2[0, r, sl]
                        v1 = gbuf2[0, r + 1, sl]
                        v2 = gbuf2[0, r + 84, sl]
                        v3 = gbuf2[0, r + 85, sl]
                        aL = (aL + (v0 != v0).astype(jnp.int32)
                              + (v1 != v1).astype(jnp.int32))
                        aR = (aR + (v2 != v2).astype(jnp.int32)
                              + (v3 != v3).astype(jnp.int32))
                        return (aL, aR)

                    aL, aR = lax.fori_loop(
                        0, 21, body,
                        (jnp.zeros((L,), jnp.int32),
                         jnp.zeros((L,), jnp.int32)))
                    cntL_l[pl.ds(ft * 128 + v * L, L)] = aL
                    cntR_l[pl.ds(ft * 128 + v * L, L)] = aR
                    totL = totL + aL
                    totR = totR + aR
            stageL[...] = jnp.full((L,), jnp.sum(totL), jnp.int32)
            stageR[...] = jnp.full((L,), jnp.sum(totR), jnp.int32)
            pltpu.sync_copy(cntL_l, sh_cntL.at[pl.ds(s * FPT, FPT)])
            pltpu.sync_copy(cntR_l, sh_cntR.at[pl.ds(s * FPT, FPT)])
            pltpu.sync_copy(stageL, sh_totL.at[s])
            pltpu.sync_copy(stageR, sh_totR.at[s])
            plsc.subcore_barrier()

            pltpu.sync_copy(sh_totL, totL_all)
            pltpu.sync_copy(sh_totR, totR_all)

            def tot_body(i, a):
                aL, aR = a
                return (aL + totL_all[i, :], aR + totR_all[i, :])

            accL, accR = lax.fori_loop(
                0, NS, tot_body,
                (jnp.zeros((L,), jnp.int32), jnp.zeros((L,), jnp.int32)),
            )
            ldv = accL <= accR

            @pl.when(s == 0)
            def _tile0():
                pltpu.sync_copy(sh_cntL, cntL_all)
                pltpu.sync_copy(sh_cntR, cntR_all)

                def comp(g, carry):
                    cl = cntL_all[pl.ds(g * L, L)]
                    cr = cntR_all[pl.ds(g * L, L)]
                    cnt = jnp.where(ldv, cl, cr)
                    m = cnt < 42
                    mi = m.astype(jnp.int32)
                    pos = carry + plsc.cumsum(mi) - mi
                    fid = g * L + lane
                    plsc.store_scatter(idx_buf, [pos], fid, mask=m)
                    return carry + plsc.all_reduce_population_count(m)

                count = lax.fori_loop(
                    0, N_FRAMES // L, comp, jnp.zeros((L,), jnp.int32))

                cnt0 = jnp.sum(jnp.where(lane == 0, count, 0))
                g0 = lax.shift_right_logical(cnt0, 4)
                pad = lane >= jnp.bitwise_and(cnt0, 15)
                plsc.store_scatter(
                    idx_buf, [g0 * L + lane],
                    jnp.zeros((L,), jnp.int32),
                    mask=jnp.logical_and(pad, g0 * L + lane < N_FRAMES))

                def zero(i, carry):
                    idx_buf[pl.ds(i * L, L)] = jnp.zeros((L,), jnp.int32)
                    return carry

                lax.fori_loop(g0 + 1, N_FRAMES // L, zero, 0)
                pltpu.sync_copy(idx_buf, oidx_hbm)
                for v in range(OUT_F // L):
                    sl = pl.ds(v * L, L)
                    idx64[sl] = idx_buf[sl]
                pltpu.sync_copy(idx64, sh_idx)

            for v in range(NGATHER):
                sl = pl.ds(v * L, L)
                lidx_sel[sl] = jnp.where(
                    ldv, tbl_v[pl.ds(384 + v * L, L)],
                    tbl_v[pl.ds(528 + v * L, L)])
                mul_sel[sl] = jnp.where(
                    ldv, jnp.full((L,), 1.0, jnp.float32),
                    plsc.bitcast(tbl_v[pl.ds(672 + v * L, L)], jnp.float32))
                add_sel[sl] = jnp.where(
                    ldv, jnp.full((L,), 0.0, jnp.float32),
                    plsc.bitcast(tbl_v[pl.ds(816 + v * L, L)], jnp.float32))

            plsc.subcore_barrier()

            pltpu.sync_copy(sh_idx, idx64)
            zero16 = jnp.zeros((L,), jnp.int32)

            def issue(j):
                p = j & 1
                fo = s * FPT3 + j
                ch_i = lax.div(fo, L)
                lane_i = lax.rem(fo, L)
                idxv = idx64[pl.ds(ch_i * L, L)]
                sel = jnp.sum(jnp.where(lane == lane_i, idxv, zero16))
                ft2 = lax.shift_right_logical(sel, 7) * 2
                fl = jnp.bitwise_and(sel, 127)
                for v in range(8):
                    idxA2[p, pl.ds(v * L, L)] = (
                        tbl_v[pl.ds(192 + v * L, L)] + ft2)
                for v in range(4):
                    idxB2[p, pl.ds(v * L, L)] = (
                        tbl_v[pl.ds(320 + v * L, L)] + ft2)
                hA = pltpu.async_copy(
                    y_hbm.at[idxA2.at[p]], gbuf2.at[p, pl.ds(0, 128)],
                    sems[p])
                hB = pltpu.async_copy(
                    y_hbm.at[idxB2.at[p]], gbuf2.at[p, pl.ds(128, 64)],
                    sems[p])
                return (hA, hB, fl)

            pend = issue(0)
            for j in range(FPT3):
                hA, hB, fl = pend
                if j + 1 < FPT3:
                    nxt = issue(j + 1)
                hA.wait()
                hB.wait()
                p = j & 1
                flv = jnp.full((L,), fl, jnp.int32)
                for v in range(NGATHER):
                    sl = pl.ds(v * L, L)
                    vals = plsc.load_gather(
                        gbuf2.at[p], [lidx_sel[sl], flv])
                    t = vals * mul_sel[sl] + add_sel[sl]
                    t = jnp.where(vals != vals, jnp.float32(0.0), t)
                    obuf[pl.ds((j * NGATHER + v) * L, L)] = t
                if j + 1 < FPT3:
                    pend = nxt
            pltpu.sync_copy(
                obuf, out1_hbm.at[pl.ds(s * FPT3 * OUT_W, FPT3 * OUT_W)])

    return _sc_kernel


def kernel(x):
    y = x.transpose(1, 0, 2).reshape(N_LM, NFT, 128, 2)
    y = y.transpose(0, 1, 3, 2).reshape(NROW, 128)
    out1, oidx = _build_sc_kernel()(y, jnp.asarray(_TBL))
    x1 = out1.reshape(OUT_F, OUT_W)[:, : 2 * N_OUT_LM].reshape(
        OUT_F, N_OUT_LM, 2)
    return (x1, oidx)


# speedup vs baseline: 62.4628x; 1.0238x over previous
import functools

import jax
import jax.numpy as jnp
import numpy as np
from jax import lax
from jax.experimental import pallas as pl
from jax.experimental.pallas import tpu as pltpu
from jax.experimental.pallas import tpu_sc as plsc

_LEFT_HAND = np.arange(468, 489)
_LEFT_POSE = np.array([502, 504, 506, 508, 510])
_LIPS = np.array([
    61, 185, 40, 39, 37, 0, 267, 269, 270, 409, 291, 146, 91, 181, 84, 17,
    314, 405, 321, 375, 78, 191, 80, 81, 82, 13, 312, 311, 310, 415, 95, 88,
    178, 87, 14, 317, 402, 318, 324, 308,
])
_RIGHT_HAND = np.arange(522, 543)
_RIGHT_POSE = np.array([503, 505, 507, 509, 511])

N_FRAMES = 4096
N_LM = 543
N_OUT_LM = 66
OUT_F = 64
NS = 16
L = 16
FPT = N_FRAMES // NS
NGATHER = 9
OUT_W = NGATHER * L
FPT3 = OUT_F // NS
NFT = N_FRAMES // 128
NROW = N_LM * 2 * NFT

_UNION_LM = np.concatenate(
    (_LIPS, _LEFT_HAND, _LEFT_POSE, _RIGHT_HAND, _RIGHT_POSE))

_HANDS = np.concatenate((_LEFT_HAND, _RIGHT_HAND))
_HTAB = np.zeros(192, np.int32)
for _h in range(42):
    for _q in range(4):
        _HTAB[_h * 4 + _q] = _HANDS[_h] * 64 + (_q & 1) + (_q >> 1) * 2
for _h in range(42):
    for _ft in range(2):
        for _c in range(2):
            _HTAB[_h * 4 + _ft * 2 + _c] = _HANDS[_h] * 64 + _ft * 2 + _c

_UB = np.zeros(192, np.int32)
for _u in range(92):
    for _c in range(2):
        _UB[2 * _u + _c] = _UNION_LM[_u] * 64 + _c

_LIDXU_L = np.pad(np.arange(132, dtype=np.int32), (0, OUT_W - 132))
_uR = np.concatenate((np.arange(40), np.arange(66, 92)))
_LIDXU_R = np.pad(
    np.stack([2 * _uR, 2 * _uR + 1], axis=1).reshape(-1).astype(np.int32),
    (0, OUT_W - 132))

_MUL_R = np.ones(OUT_W, np.float32)
_ADD_R = np.zeros(OUT_W, np.float32)
for _l in range(40, N_OUT_LM):
    _MUL_R[2 * _l] = -1.0
    _ADD_R[2 * _l] = 1.0

_TBL = np.concatenate([
    _HTAB, _UB, _LIDXU_L, _LIDXU_R,
    _MUL_R.view(np.int32), _ADD_R.view(np.int32),
]).astype(np.int32)


@functools.cache
def _build_sc_kernel():
    mesh = plsc.VectorSubcoreMesh(
        core_axis_name="c", subcore_axis_name="s", num_cores=2,
        num_subcores=NS,
    )

    @functools.partial(
        pl.kernel,
        out_type=[
            jax.ShapeDtypeStruct((OUT_F * OUT_W,), jnp.float32),
            jax.ShapeDtypeStruct((N_FRAMES,), jnp.int32),
        ],
        mesh=mesh,
        compiler_params=pltpu.CompilerParams(
            needs_layout_passes=False, use_tc_tiling_on_sc=False),
        scratch_types=[
            pltpu.VMEM((2, 192, 128), jnp.float32),
            pltpu.VMEM((2, 128), jnp.int32),
            pltpu.VMEM((2, 64), jnp.int32),
            pltpu.VMEM((960,), jnp.int32),
            pltpu.VMEM((FPT,), jnp.int32),
            pltpu.VMEM((FPT,), jnp.int32),
            pltpu.VMEM((L,), jnp.int32),
            pltpu.VMEM((L,), jnp.int32),
            pltpu.VMEM_SHARED((N_FRAMES,), jnp.int32),
            pltpu.VMEM_SHARED((N_FRAMES,), jnp.int32),
            pltpu.VMEM_SHARED((NS, L), jnp.int32),
            pltpu.VMEM_SHARED((NS, L), jnp.int32),
            pltpu.VMEM_SHARED((OUT_F,), jnp.int32),
            pltpu.VMEM((N_FRAMES,), jnp.int32),
            pltpu.VMEM((N_FRAMES,), jnp.int32),
            pltpu.VMEM((NS, L), jnp.int32),
            pltpu.VMEM((NS, L), jnp.int32),
            pltpu.VMEM((N_FRAMES,), jnp.int32),
            pltpu.VMEM((OUT_F,), jnp.int32),
            pltpu.VMEM((FPT3 * OUT_W,), jnp.float32),
            pltpu.VMEM((OUT_W,), jnp.int32),
            pltpu.VMEM((OUT_W,), jnp.float32),
            pltpu.VMEM((OUT_W,), jnp.float32),
            pltpu.SemaphoreType.DMA,
            pltpu.SemaphoreType.DMA,
        ],
    )
    def _sc_kernel(
        y_hbm, tbl_hbm,
        out1_hbm, oidx_hbm,
        gbuf2, idxA2, idxB2, tbl_v,
        cntL_l, cntR_l, stageL, stageR,
        sh_cntL, sh_cntR, sh_totL, sh_totR, sh_idx,
        cntL_all, cntR_all, totL_all, totR_all,
        idx_buf, idx64, obuf,
        lidx_sel, mul_sel, add_sel,
        sem0, sem1,
    ):
        sems = (sem0, sem1)
        c = lax.axis_index("c")
        s = lax.axis_index("s")

        @pl.when(c == 0)
        def _core0():
            lane = lax.iota(jnp.int32, L)

            pltpu.sync_copy(tbl_hbm, tbl_v)
            s4 = s * 4
            for v in range(8):
                idxA2[0, pl.ds(v * L, L)] = tbl_v[pl.ds(v * L, L)] + s4
            for v in range(4):
                idxB2[0, pl.ds(v * L, L)] = tbl_v[pl.ds(128 + v * L, L)] + s4
            h1 = pltpu.async_copy(
                y_hbm.at[idxA2.at[0]], gbuf2.at[0, pl.ds(0, 128)], sem0)
            h2 = pltpu.async_copy(
                y_hbm.at[idxB2.at[0]], gbuf2.at[0, pl.ds(128, 64)], sem0)
            h1.wait()
            h2.wait()

            totL = jnp.zeros((L,), jnp.int32)
            totR = jnp.zeros((L,), jnp.int32)
            for ft in range(2):
                for v in range(8):
                    sl = pl.ds(v * L, L)
                    ft2 = ft * 2

                    def body(h, a, ft2=ft2, sl=sl):
                        aL, aR = a
                        r = h * 4 + ft2
                        v0 = gbuf2[0, r, sl]
                        v1 = gbuf2[0, r + 1, sl]
                        v2 = gbuf2[0, r + 84, sl]
                        v3 = gbuf2[0, r + 85, sl]
                        aL = (aL + (v0 != v0).astype(jnp.int32)
                              + (v1 != v1).astype(jnp.int32))
                        aR = (aR + (v2 != v2).astype(jnp.int32)
                              + (v3 != v3).astype(jnp.int32))
                        return (aL, aR)

                    aL, aR = lax.fori_loop(
                        0, 21, body,
                        (jnp.zeros((L,), jnp.int32),
                         jnp.zeros((L,), jnp.int32)))
                    cntL_l[pl.ds(ft * 128 + v * L, L)] = aL
                    cntR_l[pl.ds(ft * 128 + v * L, L)] = aR
                    totL = totL + aL
                    totR = totR + aR
            stageL[...] = jnp.full((L,), jnp.sum(totL), jnp.int32)
            stageR[...] = jnp.full((L,), jnp.sum(totR), jnp.int32)
            pltpu.sync_copy(cntL_l, sh_cntL.at[pl.ds(s * FPT, FPT)])
            pltpu.sync_copy(cntR_l, sh_cntR.at[pl.ds(s * FPT, FPT)])
            pltpu.sync_copy(stageL, sh_totL.at[s])
            pltpu.sync_copy(stageR, sh_totR.at[s])
            plsc.subcore_barrier()

            pltpu.sync_copy(sh_totL, totL_all)
            pltpu.sync_copy(sh_totR, totR_all)

            def tot_body(i, a):
                aL, aR = a
                return (aL + totL_all[i, :], aR + totR_all[i, :])

            accL, accR = lax.fori_loop(
                0, NS, tot_body,
                (jnp.zeros((L,), jnp.int32), jnp.zeros((L,), jnp.int32)),
            )
            ldv = accL <= accR

            @pl.when(s == 0)
            def _tile0():
                pltpu.sync_copy(sh_cntL, cntL_all)
                pltpu.sync_copy(sh_cntR, cntR_all)

                def comp(g, carry):
                    b = g * 2 * L
                    cl0 = cntL_all[pl.ds(b, L)]
                    cr0 = cntR_all[pl.ds(b, L)]
                    cl1 = cntL_all[pl.ds(b + L, L)]
                    cr1 = cntR_all[pl.ds(b + L, L)]
                    m0 = jnp.where(ldv, cl0, cr0) < 42
                    m1 = jnp.where(ldv, cl1, cr1) < 42
                    mi0 = m0.astype(jnp.int32)
                    mi1 = m1.astype(jnp.int32)
                    cs0 = plsc.cumsum(mi0)
                    cs1 = plsc.cumsum(mi1)
                    p0 = plsc.all_reduce_population_count(m0)
                    p1 = plsc.all_reduce_population_count(m1)
                    pos0 = carry + cs0 - mi0
                    pos1 = carry + p0 + cs1 - mi1
                    plsc.store_scatter(idx_buf, [pos0], b + lane, mask=m0)
                    plsc.store_scatter(
                        idx_buf, [pos1], b + L + lane, mask=m1)
                    return carry + p0 + p1

                count = lax.fori_loop(
                    0, N_FRAMES // (2 * L), comp, jnp.zeros((L,), jnp.int32))

                cnt0 = jnp.sum(jnp.where(lane == 0, count, 0))
                g0 = lax.shift_right_logical(cnt0, 4)
                pad = lane >= jnp.bitwise_and(cnt0, 15)
                plsc.store_scatter(
                    idx_buf, [g0 * L + lane],
                    jnp.zeros((L,), jnp.int32),
                    mask=jnp.logical_and(pad, g0 * L + lane < N_FRAMES))

                def zero(i, carry):
                    idx_buf[pl.ds(i * L, L)] = jnp.zeros((L,), jnp.int32)
                    return carry

                lax.fori_loop(g0 + 1, N_FRAMES // L, zero, 0)
                pltpu.sync_copy(idx_buf, oidx_hbm)
                for v in range(OUT_F // L):
                    sl = pl.ds(v * L, L)
                    idx64[sl] = idx_buf[sl]
                pltpu.sync_copy(idx64, sh_idx)

            for v in range(NGATHER):
                sl = pl.ds(v * L, L)
                lidx_sel[sl] = jnp.where(
                    ldv, tbl_v[pl.ds(384 + v * L, L)],
                    tbl_v[pl.ds(528 + v * L, L)])
                mul_sel[sl] = jnp.where(
                    ldv, jnp.full((L,), 1.0, jnp.float32),
                    plsc.bitcast(tbl_v[pl.ds(672 + v * L, L)], jnp.float32))
                add_sel[sl] = jnp.where(
                    ldv, jnp.full((L,), 0.0, jnp.float32),
                    plsc.bitcast(tbl_v[pl.ds(816 + v * L, L)], jnp.float32))

            plsc.subcore_barrier()

            pltpu.sync_copy(sh_idx, idx64)
            zero16 = jnp.zeros((L,), jnp.int32)

            def issue(j):
                p = j & 1
                fo = s * FPT3 + j
                ch_i = lax.div(fo, L)
                lane_i = lax.rem(fo, L)
                idxv = idx64[pl.ds(ch_i * L, L)]
                sel = jnp.sum(jnp.where(lane == lane_i, idxv, zero16))
                ft2 = lax.shift_right_logical(sel, 7) * 2
                fl = jnp.bitwise_and(sel, 127)
                for v in range(8):
                    idxA2[p, pl.ds(v * L, L)] = (
                        tbl_v[pl.ds(192 + v * L, L)] + ft2)
                for v in range(4):
                    idxB2[p, pl.ds(v * L, L)] = (
                        tbl_v[pl.ds(320 + v * L, L)] + ft2)
                hA = pltpu.async_copy(
                    y_hbm.at[idxA2.at[p]], gbuf2.at[p, pl.ds(0, 128)],
                    sems[p])
                hB = pltpu.async_copy(
                    y_hbm.at[idxB2.at[p]], gbuf2.at[p, pl.ds(128, 64)],
                    sems[p])
                return (hA, hB, fl)

            pend = issue(0)
            for j in range(FPT3):
                hA, hB, fl = pend
                if j + 1 < FPT3:
                    nxt = issue(j + 1)
                hA.wait()
                hB.wait()
                p = j & 1
                flv = jnp.full((L,), fl, jnp.int32)
                for v in range(NGATHER):
                    sl = pl.ds(v * L, L)
                    vals = plsc.load_gather(
                        gbuf2.at[p], [lidx_sel[sl], flv])
                    t = vals * mul_sel[sl] + add_sel[sl]
                    t = jnp.where(vals != vals, jnp.float32(0.0), t)
                    obuf[pl.ds((j * NGATHER + v) * L, L)] = t
                if j + 1 < FPT3:
                    pend = nxt
            pltpu.sync_copy(
                obuf, out1_hbm.at[pl.ds(s * FPT3 * OUT_W, FPT3 * OUT_W)])

    return _sc_kernel


def kernel(x):
    y = x.transpose(1, 0, 2).reshape(N_LM, NFT, 128, 2)
    y = y.transpose(0, 1, 3, 2).reshape(NROW, 128)
    out1, oidx = _build_sc_kernel()(y, jnp.asarray(_TBL))
    x1 = out1.reshape(OUT_F, OUT_W)[:, : 2 * N_OUT_LM].reshape(
        OUT_F, N_OUT_LM, 2)
    return (x1, oidx)


# speedup vs baseline: 62.7721x; 1.0050x over previous
import functools

import jax
import jax.numpy as jnp
import numpy as np
from jax import lax
from jax.experimental import pallas as pl
from jax.experimental.pallas import tpu as pltpu
from jax.experimental.pallas import tpu_sc as plsc

_LEFT_HAND = np.arange(468, 489)
_LEFT_POSE = np.array([502, 504, 506, 508, 510])
_LIPS = np.array([
    61, 185, 40, 39, 37, 0, 267, 269, 270, 409, 291, 146, 91, 181, 84, 17,
    314, 405, 321, 375, 78, 191, 80, 81, 82, 13, 312, 311, 310, 415, 95, 88,
    178, 87, 14, 317, 402, 318, 324, 308,
])
_RIGHT_HAND = np.arange(522, 543)
_RIGHT_POSE = np.array([503, 505, 507, 509, 511])

N_FRAMES = 4096
N_LM = 543
N_OUT_LM = 66
OUT_F = 64
NS = 16
L = 16
FPT = N_FRAMES // NS
NGATHER = 9
OUT_W = NGATHER * L
FPT3 = OUT_F // NS
NFT = N_FRAMES // 128
NROW = N_LM * 2 * NFT

_UNION_LM = np.concatenate(
    (_LIPS, _LEFT_HAND, _LEFT_POSE, _RIGHT_HAND, _RIGHT_POSE))

_HANDS = np.concatenate((_LEFT_HAND, _RIGHT_HAND))
_HTAB = np.zeros(192, np.int32)
for _h in range(42):
    for _q in range(4):
        _HTAB[_h * 4 + _q] = _HANDS[_h] * 64 + (_q & 1) + (_q >> 1) * 2
for _h in range(42):
    for _ft in range(2):
        for _c in range(2):
            _HTAB[_h * 4 + _ft * 2 + _c] = _HANDS[_h] * 64 + _ft * 2 + _c

_UB = np.zeros(192, np.int32)
for _u in range(92):
    for _c in range(2):
        _UB[2 * _u + _c] = _UNION_LM[_u] * 64 + _c

_LIDXU_L = np.pad(np.arange(132, dtype=np.int32), (0, OUT_W - 132))
_uR = np.concatenate((np.arange(40), np.arange(66, 92)))
_LIDXU_R = np.pad(
    np.stack([2 * _uR, 2 * _uR + 1], axis=1).reshape(-1).astype(np.int32),
    (0, OUT_W - 132))

_MUL_R = np.ones(OUT_W, np.float32)
_ADD_R = np.zeros(OUT_W, np.float32)
for _l in range(40, N_OUT_LM):
    _MUL_R[2 * _l] = -1.0
    _ADD_R[2 * _l] = 1.0

_TBL = np.concatenate([
    _HTAB, _UB, _LIDXU_L, _LIDXU_R,
    _MUL_R.view(np.int32), _ADD_R.view(np.int32),
]).astype(np.int32)


@functools.cache
def _build_sc_kernel():
    mesh = plsc.VectorSubcoreMesh(
        core_axis_name="c", subcore_axis_name="s", num_cores=2,
        num_subcores=NS,
    )

    @functools.partial(
        pl.kernel,
        out_type=[
            jax.ShapeDtypeStruct((OUT_F * OUT_W,), jnp.float32),
            jax.ShapeDtypeStruct((N_FRAMES,), jnp.int32),
        ],
        mesh=mesh,
        compiler_params=pltpu.CompilerParams(
            needs_layout_passes=False, use_tc_tiling_on_sc=False),
        scratch_types=[
            pltpu.VMEM((2, 192, 128), jnp.float32),
            pltpu.VMEM((2, 128), jnp.int32),
            pltpu.VMEM((2, 64), jnp.int32),
            pltpu.VMEM((960,), jnp.int32),
            pltpu.VMEM((FPT,), jnp.int32),
            pltpu.VMEM((FPT,), jnp.int32),
            pltpu.VMEM((L,), jnp.int32),
            pltpu.VMEM((L,), jnp.int32),
            pltpu.VMEM_SHARED((N_FRAMES,), jnp.int32),
            pltpu.VMEM_SHARED((N_FRAMES,), jnp.int32),
            pltpu.VMEM_SHARED((NS, L), jnp.int32),
            pltpu.VMEM_SHARED((NS, L), jnp.int32),
            pltpu.VMEM_SHARED((OUT_F,), jnp.int32),
            pltpu.VMEM((N_FRAMES,), jnp.int32),
            pltpu.VMEM((N_FRAMES,), jnp.int32),
            pltpu.VMEM((NS, L), jnp.int32),
            pltpu.VMEM((NS, L), jnp.int32),
            pltpu.VMEM((N_FRAMES,), jnp.int32),
            pltpu.VMEM((OUT_F,), jnp.int32),
            pltpu.VMEM((FPT3 * OUT_W,), jnp.float32),
            pltpu.VMEM((OUT_W,), jnp.int32),
            pltpu.VMEM((OUT_W,), jnp.float32),
            pltpu.VMEM((OUT_W,), jnp.float32),
            pltpu.SemaphoreType.DMA,
            pltpu.SemaphoreType.DMA,
        ],
    )
    def _sc_kernel(
        y_hbm, tbl_hbm,
        out1_hbm, oidx_hbm,
        gbuf2, idxA2, idxB2, tbl_v,
        cntL_l, cntR_l, stageL, stageR,
        sh_cntL, sh_cntR, sh_totL, sh_totR, sh_idx,
        cntL_all, cntR_all, totL_all, totR_all,
        idx_buf, idx64, obuf,
        lidx_sel, mul_sel, add_sel,
        sem0, sem1,
    ):
        sems = (sem0, sem1)
        c = lax.axis_index("c")
        s = lax.axis_index("s")

        @pl.when(c == 0)
        def _core0():
            lane = lax.iota(jnp.int32, L)

            pltpu.sync_copy(tbl_hbm, tbl_v)
            s4 = s * 4
            for v in range(8):
                idxA2[0, pl.ds(v * L, L)] = tbl_v[pl.ds(v * L, L)] + s4
            for v in range(4):
                idxB2[0, pl.ds(v * L, L)] = tbl_v[pl.ds(128 + v * L, L)] + s4
            h1 = pltpu.async_copy(
                y_hbm.at[idxA2.at[0]], gbuf2.at[0, pl.ds(0, 128)], sem0)
            h2 = pltpu.async_copy(
                y_hbm.at[idxB2.at[0]], gbuf2.at[0, pl.ds(128, 64)], sem0)
            h1.wait()
            h2.wait()

            totL = jnp.zeros((L,), jnp.int32)
            totR = jnp.zeros((L,), jnp.int32)
            for ft in range(2):
                for v in range(8):
                    sl = pl.ds(v * L, L)
                    ft2 = ft * 2

                    def body(h, a, ft2=ft2, sl=sl):
                        aL, aR = a
                        r = h * 4 + ft2
                        v0 = gbuf2[0, r, sl]
                        v1 = gbuf2[0, r + 1, sl]
                        v2 = gbuf2[0, r + 84, sl]
                        v3 = gbuf2[0, r + 85, sl]
                        aL = (aL + (v0 != v0).astype(jnp.int32)
                              + (v1 != v1).astype(jnp.int32))
                        aR = (aR + (v2 != v2).astype(jnp.int32)
                              + (v3 != v3).astype(jnp.int32))
                        return (aL, aR)

                    aL, aR = lax.fori_loop(
                        0, 21, body,
                        (jnp.zeros((L,), jnp.int32),
                         jnp.zeros((L,), jnp.int32)))
                    cntL_l[pl.ds(ft * 128 + v * L, L)] = aL
                    cntR_l[pl.ds(ft * 128 + v * L, L)] = aR
                    totL = totL + aL
                    totR = totR + aR
            stageL[...] = jnp.full((L,), jnp.sum(totL), jnp.int32)
            stageR[...] = jnp.full((L,), jnp.sum(totR), jnp.int32)
            pltpu.sync_copy(cntL_l, sh_cntL.at[pl.ds(s * FPT, FPT)])
            pltpu.sync_copy(cntR_l, sh_cntR.at[pl.ds(s * FPT, FPT)])
            pltpu.sync_copy(stageL, sh_totL.at[s])
            pltpu.sync_copy(stageR, sh_totR.at[s])
            plsc.subcore_barrier()

            pltpu.sync_copy(sh_totL, totL_all)
            pltpu.sync_copy(sh_totR, totR_all)

            def tot_body(i, a):
                aL, aR = a
                return (aL + totL_all[i, :], aR + totR_all[i, :])

            accL, accR = lax.fori_loop(
                0, NS, tot_body,
                (jnp.zeros((L,), jnp.int32), jnp.zeros((L,), jnp.int32)),
            )
            ldv = accL <= accR

            @pl.when(s == 0)
            def _tile0():
                pltpu.sync_copy(sh_cntL, cntL_all)
                pltpu.sync_copy(sh_cntR, cntR_all)

                def comp(g, carry):
                    b = g * 2 * L
                    cl0 = cntL_all[pl.ds(b, L)]
                    cr0 = cntR_all[pl.ds(b, L)]
                    cl1 = cntL_all[pl.ds(b + L, L)]
                    cr1 = cntR_all[pl.ds(b + L, L)]
                    m0 = jnp.where(ldv, cl0, cr0) < 42
                    m1 = jnp.where(ldv, cl1, cr1) < 42
                    mi0 = m0.astype(jnp.int32)
                    mi1 = m1.astype(jnp.int32)
                    cs0 = plsc.cumsum(mi0)
                    cs1 = plsc.cumsum(mi1)
                    p0 = plsc.all_reduce_population_count(m0)
                    p1 = plsc.all_reduce_population_count(m1)
                    pos0 = carry + cs0 - mi0
                    pos1 = carry + p0 + cs1 - mi1
                    plsc.store_scatter(idx_buf, [pos0], b + lane, mask=m0)
                    plsc.store_scatter(
                        idx_buf, [pos1], b + L + lane, mask=m1)
                    return carry + p0 + p1

                count = lax.fori_loop(
                    0, N_FRAMES // (2 * L), comp, jnp.zeros((L,), jnp.int32))

                cnt0 = jnp.sum(jnp.where(lane == 0, count, 0))
                g0 = lax.shift_right_logical(cnt0, 4)
                pad = lane >= jnp.bitwise_and(cnt0, 15)
                plsc.store_scatter(
                    idx_buf, [g0 * L + lane],
                    jnp.zeros((L,), jnp.int32),
                    mask=jnp.logical_and(pad, g0 * L + lane < N_FRAMES))

                def zero(i, carry):
                    idx_buf[pl.ds(i * L, L)] = jnp.zeros((L,), jnp.int32)
                    return carry

                lax.fori_loop(g0 + 1, N_FRAMES // L, zero, 0)
                for v in range(OUT_F // L):
                    sl = pl.ds(v * L, L)
                    idx64[sl] = idx_buf[sl]
                pltpu.sync_copy(idx64, sh_idx)

            for v in range(NGATHER):
                sl = pl.ds(v * L, L)
                lidx_sel[sl] = jnp.where(
                    ldv, tbl_v[pl.ds(384 + v * L, L)],
                    tbl_v[pl.ds(528 + v * L, L)])
                mul_sel[sl] = jnp.where(
                    ldv, jnp.full((L,), 1.0, jnp.float32),
                    plsc.bitcast(tbl_v[pl.ds(672 + v * L, L)], jnp.float32))
                add_sel[sl] = jnp.where(
                    ldv, jnp.full((L,), 0.0, jnp.float32),
                    plsc.bitcast(tbl_v[pl.ds(816 + v * L, L)], jnp.float32))

            plsc.subcore_barrier()

            @pl.when(s == 0)
            def _oidx():
                pltpu.sync_copy(idx_buf, oidx_hbm)

            pltpu.sync_copy(sh_idx, idx64)
            zero16 = jnp.zeros((L,), jnp.int32)

            def issue(j):
                p = j & 1
                fo = s * FPT3 + j
                ch_i = lax.div(fo, L)
                lane_i = lax.rem(fo, L)
                idxv = idx64[pl.ds(ch_i * L, L)]
                sel = jnp.sum(jnp.where(lane == lane_i, idxv, zero16))
                ft2 = lax.shift_right_logical(sel, 7) * 2
                fl = jnp.bitwise_and(sel, 127)
                for v in range(8):
                    idxA2[p, pl.ds(v * L, L)] = (
                        tbl_v[pl.ds(192 + v * L, L)] + ft2)
                for v in range(4):
                    idxB2[p, pl.ds(v * L, L)] = (
                        tbl_v[pl.ds(320 + v * L, L)] + ft2)
                hA = pltpu.async_copy(
                    y_hbm.at[idxA2.at[p]], gbuf2.at[p, pl.ds(0, 128)],
                    sems[p])
                hB = pltpu.async_copy(
                    y_hbm.at[idxB2.at[p]], gbuf2.at[p, pl.ds(128, 64)],
                    sems[p])
                return (hA, hB, fl)

            pend = issue(0)
            for j in range(FPT3):
                hA, hB, fl = pend
                if j + 1 < FPT3:
                    nxt = issue(j + 1)
                hA.wait()
                hB.wait()
                p = j & 1
                flv = jnp.full((L,), fl, jnp.int32)
                for v in range(NGATHER):
                    sl = pl.ds(v * L, L)
                    vals = plsc.load_gather(
                        gbuf2.at[p], [lidx_sel[sl], flv])
                    t = vals * mul_sel[sl] + add_sel[sl]
                    t = jnp.where(vals != vals, jnp.float32(0.0), t)
                    obuf[pl.ds((j * NGATHER + v) * L, L)] = t
                if j + 1 < FPT3:
                    pend = nxt
            pltpu.sync_copy(
                obuf, out1_hbm.at[pl.ds(s * FPT3 * OUT_W, FPT3 * OUT_W)])

    return _sc_kernel


def kernel(x):
    y = x.transpose(1, 0, 2).reshape(N_LM, NFT, 128, 2)
    y = y.transpose(0, 1, 3, 2).reshape(NROW, 128)
    out1, oidx = _build_sc_kernel()(y, jnp.asarray(_TBL))
    x1 = out1.reshape(OUT_F, OUT_W)[:, : 2 * N_OUT_LM].reshape(
        OUT_F, N_OUT_LM, 2)
    return (x1, oidx)
